# baseline (speedup 1.0000x reference)
import jax
import jax.numpy as jnp
from jax import lax
from jax.experimental import pallas as pl
from jax.experimental.pallas import tpu as pltpu
from jax.experimental.pallas import tpu_sc as plsc

NSYS = 4
NA = 48
NM = 5
NP = 64
ROW = NM * NP
ROWP = 384
BW = 640
BWR = 576
NKEY = NSYS * NA * NA
STOT = 9216
NORB = 3456

KT = 512
ST = 512
NKT = NKEY // KT
NST = STOT // ST

NC = 2
NS = 16
NW = NC * NS
KPW = NKEY // NW
CHUNK = 96
WCH = KPW // CHUNK


def _w_body(cg_s, v_r, w_ref, m_ref):
    @pl.when(pl.program_id(0) == 0)
    def _build_m():
        rowi = lax.broadcasted_iota(jnp.int32, (ROWP, BW), 0)
        coli = lax.broadcasted_iota(jnp.int32, (ROWP, BW), 1)
        r_i = coli // 24
        c_i = coli - r_i * 24
        n1p = r_i // 3
        a_i = r_i - n1p * 3
        n2p = c_i // 3
        b_i = c_i - n2p * 3
        tgt = n1p * 8 + n2p
        m = jnp.zeros((ROWP, BW), jnp.float32)
        for a0 in range(3):
            for b0 in range(3):
                for m0 in range(NM):
                    mask = ((rowi == tgt + m0 * 64) & (a_i == a0)
                            & (b_i == b0) & (r_i < 24))
                    m = m + jnp.where(
                        mask, cg_s[(a0 * 3 + b0) * NM + m0], 0.0)
        m_ref[...] = m

    w_ref[0] = jnp.dot(v_r[0], m_ref[...],
                       preferred_element_type=jnp.float32)


def _tc_transform(values3, cg_flat):
    return pl.pallas_call(
        _w_body,
        grid=(NST,),
        in_specs=[
            pl.BlockSpec(memory_space=pltpu.SMEM),
            pl.BlockSpec((1, ST, ROWP), lambda st: (st, 0, 0)),
        ],
        out_specs=pl.BlockSpec((1, ST, BW), lambda st: (st, 0, 0)),
        out_shape=jax.ShapeDtypeStruct((NST, ST, BW), jnp.float32),
        scratch_shapes=[pltpu.VMEM((ROWP, BW), jnp.float32)],
    )(cg_flat, values3)


def _seg_body(sys_r, i1_r, i2_r, w_r, acc_ref):
    kt = pl.program_id(0)
    st = pl.program_id(1)
    s_v = sys_r[0, 0]
    i1v = i1_r[0, 0]
    i2v = i2_r[0, 0]
    k1 = (s_v * NA + i1v) * NA + i2v
    col = lax.broadcasted_iota(jnp.int32, (ST, KT), 1) + kt * KT
    e1 = (k1[:, None] == col).astype(jnp.float32)
    d1 = lax.dot_general(e1, w_r[0], (((0,), (0,)), ((), ())),
                         preferred_element_type=jnp.float32)

    @pl.when(st == 0)
    def _init():
        acc_ref[0] = d1

    @pl.when(st != 0)
    def _accum():
        acc_ref[0] += d1


def _tc_segment_sum(w3, sys3, i13, i23):
    return pl.pallas_call(
        _seg_body,
        grid=(NKT, NST),
        in_specs=[
            pl.BlockSpec((1, 1, ST), lambda kt, st: (st, 0, 0)),
            pl.BlockSpec((1, 1, ST), lambda kt, st: (st, 0, 0)),
            pl.BlockSpec((1, 1, ST), lambda kt, st: (st, 0, 0)),
            pl.BlockSpec((1, ST, BW), lambda kt, st: (st, 0, 0)),
        ],
        out_specs=pl.BlockSpec((1, KT, BW), lambda kt, st: (kt, 0, 0)),
        out_shape=jax.ShapeDtypeStruct((NKT, KT, BW), jnp.float32),
    )(sys3, i13, i23, w3)


def _perm_body(g_hbm, gt_out, vals_v, tidx_v):
    c = lax.axis_index("c")
    s = lax.axis_index("s")
    w = c * NS + s
    iota16 = lax.iota(jnp.int32, 16)
    for j in range(WCH):
        for t in range(CHUNK // 16):
            gi0, rem0 = divmod(t * 16, NA)
            g48 = w * (KPW // NA) + j * (CHUNK // NA) + gi0
            sys_j = g48 // NA
            a1_j = g48 - sys_j * NA
            base_v = jnp.full((16,), sys_j * (NA * NA) + a1_j, jnp.int32)
            tidx_v[j, pl.ds(t * 16, 16)] = base_v + (rem0 + iota16) * NA
        pltpu.sync_copy(g_hbm.at[pl.ds(w * KPW + j * CHUNK, CHUNK)], vals_v)
        pltpu.sync_copy(vals_v, gt_out.at[tidx_v.at[j]])


def _sc_permute(g2):
    run = pl.kernel(
        _perm_body,
        out_type=jax.ShapeDtypeStruct((NKEY, BW), jnp.float32),
        mesh=plsc.VectorSubcoreMesh(core_axis_name="c", subcore_axis_name="s",
                                    num_cores=NC, num_subcores=NS),
        scratch_types=[
            pltpu.VMEM((CHUNK, BW), jnp.float32),
            pltpu.VMEM((WCH, CHUNK), jnp.int32),
        ],
    )
    return run(g2)


def _tc_body(gtA, gB, out_ref):
    a3 = gtA[0, 0][:, 0:BWR].reshape(NA, 24, 24)
    b3 = gB[0, 0][:, 0:BWR].reshape(NA, 24, 24).transpose(0, 2, 1)
    payload = 0.5 * (a3 + b3)
    rows = jnp.concatenate(
        [jnp.zeros((NA, 8, 24), jnp.float32), payload,
         jnp.zeros((NA, 40, 24), jnp.float32)], axis=1)
    full = jnp.concatenate(
        [jnp.zeros((NA, 72, 8), jnp.float32), rows,
         jnp.zeros((NA, 72, 40), jnp.float32)], axis=2)
    out_ref[0, :, :, 0, 0, :] = full


def _tc_build(gt6, g6):
    return pl.pallas_call(
        _tc_body,
        grid=(NSYS, NA),
        in_specs=[
            pl.BlockSpec((1, 1, NA, BW), lambda s, a: (s, a, 0, 0)),
            pl.BlockSpec((1, 1, NA, BW), lambda s, a: (s, a, 0, 0)),
        ],
        out_specs=pl.BlockSpec((1, NA, 72, 1, 1, 72),
                               lambda s, a: (s, 0, 0, a, 0, 0)),
        out_shape=jax.ShapeDtypeStruct((NSYS, NA, 72, NA, 1, 72),
                                       jnp.float32),
    )(gt6, g6)


def kernel(values, cg, sys_idx, i1, i2):
    values3 = jnp.pad(values.reshape(STOT, ROW),
                      ((0, 0), (0, ROWP - ROW))).reshape(NST, ST, ROWP)
    sys3 = sys_idx.astype(jnp.int32).reshape(NST, 1, ST)
    i13 = i1.astype(jnp.int32).reshape(NST, 1, ST)
    i23 = i2.astype(jnp.int32).reshape(NST, 1, ST)
    w3 = _tc_transform(values3, cg.reshape(3 * 3 * NM))
    g = _tc_segment_sum(w3, sys3, i13, i23).reshape(NKEY, BW)
    gt = _sc_permute(g)
    g6 = g.reshape(NSYS, NA, NA, BW)
    gt6 = gt.reshape(NSYS, NA, NA, BW)
    h = _tc_build(gt6, g6)
    return h.reshape(NSYS, NORB, NORB)


# speedup vs baseline: 3.1149x; 3.1149x over previous
import jax
import jax.numpy as jnp
from jax import lax
from jax.experimental import pallas as pl
from jax.experimental.pallas import tpu as pltpu
from jax.experimental.pallas import tpu_sc as plsc

NSYS = 4
NA = 48
NM = 5
NP = 64
ROW = NM * NP
ROWP = 384
BW = 640
BWR = 576
NKEY = NSYS * NA * NA
STOT = 9216
NORB = 3456

KT = 512
ST = 512
NKT = NKEY // KT
NST = STOT // ST

NC = 2
NS = 16
NW = NC * NS
KPW = NKEY // NW
CHUNK = 96
WCH = KPW // CHUNK


def _w_body(cg_s, v_r, w_ref, m_ref):
    @pl.when(pl.program_id(0) == 0)
    def _build_m():
        rowi = lax.broadcasted_iota(jnp.int32, (ROW, BW), 0)
        coli = lax.broadcasted_iota(jnp.int32, (ROW, BW), 1)
        r_i = coli // 24
        c_i = coli - r_i * 24
        n1p = r_i // 3
        a_i = r_i - n1p * 3
        n2p = c_i // 3
        b_i = c_i - n2p * 3
        tgt = n1p * 8 + n2p
        m = jnp.zeros((ROW, BW), jnp.float32)
        for a0 in range(3):
            for b0 in range(3):
                for m0 in range(NM):
                    mask = ((rowi == tgt + m0 * 64) & (a_i == a0)
                            & (b_i == b0) & (r_i < 24))
                    m = m + jnp.where(
                        mask, cg_s[(a0 * 3 + b0) * NM + m0], 0.0)
        m_ref[...] = m

    w_ref[0] = jnp.dot(v_r[0], m_ref[...],
                       preferred_element_type=jnp.float32)


def _tc_transform(values3, cg_flat):
    return pl.pallas_call(
        _w_body,
        grid=(NST,),
        in_specs=[
            pl.BlockSpec(memory_space=pltpu.SMEM),
            pl.BlockSpec((1, ST, ROW), lambda st: (st, 0, 0)),
        ],
        out_specs=pl.BlockSpec((1, ST, BW), lambda st: (st, 0, 0)),
        out_shape=jax.ShapeDtypeStruct((NST, ST, BW), jnp.float32),
        scratch_shapes=[pltpu.VMEM((ROW, BW), jnp.float32)],
    )(cg_flat, values3)


def _seg_body(sys_r, i1_r, i2_r, w_r, acc_ref):
    st = pl.program_id(0)
    s_v = sys_r[0, 0]
    i1v = i1_r[0, 0]
    i2v = i2_r[0, 0]
    k1 = (s_v * NA + i1v) * NA + i2v
    w = w_r[0]
    for ktc in range(NKT):
        col = lax.broadcasted_iota(jnp.int32, (ST, KT), 1) + ktc * KT
        e1 = (k1[:, None] == col).astype(jnp.float32)
        d1 = lax.dot_general(e1, w, (((0,), (0,)), ((), ())),
                             preferred_element_type=jnp.float32)

        @pl.when(st == 0)
        def _init():
            acc_ref[ktc] = d1

        @pl.when(st != 0)
        def _accum():
            acc_ref[ktc] += d1


def _tc_segment_sum(w3, sys3, i13, i23):
    return pl.pallas_call(
        _seg_body,
        grid=(NST,),
        in_specs=[
            pl.BlockSpec((1, 1, ST), lambda st: (st, 0, 0)),
            pl.BlockSpec((1, 1, ST), lambda st: (st, 0, 0)),
            pl.BlockSpec((1, 1, ST), lambda st: (st, 0, 0)),
            pl.BlockSpec((1, ST, BW), lambda st: (st, 0, 0)),
        ],
        out_specs=pl.BlockSpec((NKT, KT, BW), lambda st: (0, 0, 0)),
        out_shape=jax.ShapeDtypeStruct((NKT, KT, BW), jnp.float32),
    )(sys3, i13, i23, w3)


def _perm_body(g_hbm, gt_out, vals_v, tidx_v):
    c = lax.axis_index("c")
    s = lax.axis_index("s")
    w = c * NS + s
    iota16 = lax.iota(jnp.int32, 16)
    for j in range(WCH):
        for t in range(CHUNK // 16):
            gi0, rem0 = divmod(t * 16, NA)
            g48 = w * (KPW // NA) + j * (CHUNK // NA) + gi0
            sys_j = g48 // NA
            a1_j = g48 - sys_j * NA
            base_v = jnp.full((16,), sys_j * (NA * NA) + a1_j, jnp.int32)
            tidx_v[j, pl.ds(t * 16, 16)] = base_v + (rem0 + iota16) * NA
        pltpu.sync_copy(g_hbm.at[pl.ds(w * KPW + j * CHUNK, CHUNK)], vals_v)
        pltpu.sync_copy(vals_v, gt_out.at[tidx_v.at[j]])


def _sc_permute(g2):
    run = pl.kernel(
        _perm_body,
        out_type=jax.ShapeDtypeStruct((NKEY, BW), jnp.float32),
        mesh=plsc.VectorSubcoreMesh(core_axis_name="c", subcore_axis_name="s",
                                    num_cores=NC, num_subcores=NS),
        scratch_types=[
            pltpu.VMEM((CHUNK, BW), jnp.float32),
            pltpu.VMEM((WCH, CHUNK), jnp.int32),
        ],
    )
    return run(g2)


def _tc_body(gA, gtB, out_ref):
    a3 = gA[0, 0][:, 0:BWR].reshape(NA, 24, 24).transpose(1, 0, 2)
    b3 = gtB[0, 0][:, 0:BWR].reshape(NA, 24, 24).transpose(2, 0, 1)
    payload = 0.5 * (a3 + b3)
    padded = jnp.concatenate(
        [jnp.zeros((24, NA, 8), jnp.float32), payload,
         jnp.zeros((24, NA, 40), jnp.float32)], axis=2).reshape(24, NORB)
    out_ref[0, 0] = jnp.zeros((72, NORB), jnp.float32)
    out_ref[0, 0, 8:32, :] = padded


def _tc_build(g6, gt6):
    return pl.pallas_call(
        _tc_body,
        grid=(NSYS, NA),
        in_specs=[
            pl.BlockSpec((1, 1, NA, BW), lambda s, a: (s, a, 0, 0)),
            pl.BlockSpec((1, 1, NA, BW), lambda s, a: (s, a, 0, 0)),
        ],
        out_specs=pl.BlockSpec((1, 1, 72, NORB), lambda s, a: (s, a, 0, 0)),
        out_shape=jax.ShapeDtypeStruct((NSYS, NA, 72, NORB), jnp.float32),
    )(g6, gt6)


def kernel(values, cg, sys_idx, i1, i2):
    values3 = values.reshape(NST, ST, ROW)
    sys3 = sys_idx.astype(jnp.int32).reshape(NST, 1, ST)
    i13 = i1.astype(jnp.int32).reshape(NST, 1, ST)
    i23 = i2.astype(jnp.int32).reshape(NST, 1, ST)
    w3 = _tc_transform(values3, cg.reshape(3 * 3 * NM))
    g = _tc_segment_sum(w3, sys3, i13, i23).reshape(NKEY, BW)
    gt = _sc_permute(g)
    g6 = g.reshape(NSYS, NA, NA, BW)
    gt6 = gt.reshape(NSYS, NA, NA, BW)
    h = _tc_build(g6, gt6)
    return h.reshape(NSYS, NORB, NORB)
